# baseline (speedup 1.0000x reference)
import jax
import jax.numpy as jnp
from jax.experimental import pallas as pl
from jax.experimental.pallas import tpu as pltpu

_POS_W = 2.0
_BLOCK_ROWS = 64


def _reduce_body(out_ref, tgt_ref, acc_ref):
    i = pl.program_id(0)
    o = out_ref[...]
    t = tgt_ref[...]
    w = jnp.where(t > 0, -_POS_W * o, jnp.where(t < 0, o, 0.0))
    p = jnp.sum(w)

    @pl.when(i == 0)
    def _():
        acc_ref[0, 0] = 0.0

    acc_ref[0, 0] += p


def kernel(output, target):
    rows, cols = output.shape
    res = pl.pallas_call(
        _reduce_body,
        grid=(rows // _BLOCK_ROWS,),
        in_specs=[
            pl.BlockSpec((_BLOCK_ROWS, cols), lambda i: (i, 0)),
            pl.BlockSpec((_BLOCK_ROWS, cols), lambda i: (i, 0)),
        ],
        out_specs=pl.BlockSpec(
            (1, 1), lambda i: (0, 0), memory_space=pltpu.SMEM
        ),
        out_shape=jax.ShapeDtypeStruct((1, 1), jnp.float32),
    )(output, target)
    return res[0, 0]
